# baseline (speedup 1.0000x reference)
import functools

import jax
import jax.numpy as jnp
from jax import lax
from jax.experimental import pallas as pl
from jax.experimental.pallas import tpu as pltpu
from jax.experimental.pallas import tpu_sc as plsc

NC, NS, L = 2, 16, 16
NW = NC * NS


@functools.partial(jax.jit, static_argnums=(3, 4, 5))
def _sc_embed(x2d, table, pos, B, S, D):
    N = B * S
    HALF = S // 2
    PER_W = N // NW
    SEQ_PER_W = PER_W // S

    NBUF = 3
    mesh = plsc.VectorSubcoreMesh(core_axis_name="c", subcore_axis_name="s")

    @functools.partial(
        pl.kernel,
        mesh=mesh,
        out_type=jax.ShapeDtypeStruct((N, D), jnp.float32),
        scratch_types=[
            pltpu.VMEM((2 * SEQ_PER_W, HALF), jnp.int32),
            pltpu.VMEM((S, D), jnp.float32),
            pltpu.VMEM((NBUF, S, D), jnp.float32),
            pltpu.SemaphoreType.DMA((NBUF,)),
            pltpu.SemaphoreType.DMA((NBUF,)),
        ],
    )
    def body(x_hbm, tab_hbm, pos_hbm, out_hbm, idx_v, pos_v, buf_v, gsem, ssem):
        wid = lax.axis_index("s") * NC + lax.axis_index("c")
        pltpu.sync_copy(x_hbm.at[pl.ds(wid * 2 * SEQ_PER_W, 2 * SEQ_PER_W)], idx_v)
        pltpu.sync_copy(pos_hbm.at[pl.ds(0, S)], pos_v)
        out_base = wid * PER_W

        def gather(c, slot):
            return (
                pltpu.make_async_copy(
                    tab_hbm.at[idx_v.at[2 * c]],
                    buf_v.at[slot, pl.ds(0, HALF)], gsem.at[slot]),
                pltpu.make_async_copy(
                    tab_hbm.at[idx_v.at[2 * c + 1]],
                    buf_v.at[slot, pl.ds(HALF, HALF)], gsem.at[slot]),
            )

        def scatter(c, slot):
            return pltpu.make_async_copy(
                buf_v.at[slot], out_hbm.at[pl.ds(out_base + c * S, S)],
                ssem.at[slot])

        g1, g2 = gather(0, 0)
        g1.start()
        g2.start()

        def seq_body(c, carry):
            slot = c % NBUF
            nxt = c + 1
            nslot = nxt % NBUF

            @pl.when(nxt < SEQ_PER_W)
            def _():
                @pl.when(c >= NBUF - 1)
                def _():
                    scatter(c - (NBUF - 1), nslot).wait()
                n1, n2 = gather(nxt, nslot)
                n1.start()
                n2.start()

            g1, g2 = gather(c, slot)
            g1.wait()
            g2.wait()

            def add_row(r, carry2):
                for j in range(D // L):
                    sl = pl.ds(j * L, L)
                    buf_v[slot, r, sl] = buf_v[slot, r, sl] + pos_v[r, sl]
                return carry2

            lax.fori_loop(0, S, add_row, 0)
            scatter(c, slot).start()
            return carry

        lax.fori_loop(0, SEQ_PER_W, seq_body, 0)
        for b in range(NBUF):
            scatter(SEQ_PER_W - NBUF + b, (SEQ_PER_W - NBUF + b) % NBUF).wait()

    return body(x2d, table, pos)


def kernel(x, embed_table, pos_table):
    B, S = x.shape
    D = embed_table.shape[1]
    x2d = x.reshape(B * S // (S // 2), S // 2)
    out = _sc_embed(x2d, embed_table, pos_table, B, S, D)
    return out.reshape(B, S, D)


# speedup vs baseline: 1.6848x; 1.6848x over previous
import functools

import jax
import jax.numpy as jnp
from jax import lax
from jax.experimental import pallas as pl
from jax.experimental.pallas import tpu as pltpu
from jax.experimental.pallas import tpu_sc as plsc

NC, NS, L = 2, 16, 16
NW = NC * NS


@functools.partial(jax.jit, static_argnums=(3, 4, 5))
def _sc_embed(x2d, table, pos, B, S, D):
    N = B * S
    HALF = S // 2
    PER_W = N // NW
    SEQ_PER_W = PER_W // S

    mesh = plsc.VectorSubcoreMesh(core_axis_name="c", subcore_axis_name="s")

    @functools.partial(
        pl.kernel,
        mesh=mesh,
        out_type=jax.ShapeDtypeStruct((N, D), jnp.float32),
        scratch_types=[
            pltpu.VMEM((2 * SEQ_PER_W, HALF), jnp.int32),
            pltpu.VMEM((S, D), jnp.float32),
            pltpu.VMEM((S, D), jnp.float32),
            pltpu.SemaphoreType.DMA,
        ],
    )
    def body(x_hbm, tab_hbm, pos_hbm, out_hbm, idx_v, pos_v, buf_v, sem):
        wid = lax.axis_index("s") * NC + lax.axis_index("c")
        pltpu.sync_copy(x_hbm.at[pl.ds(wid * 2 * SEQ_PER_W, 2 * SEQ_PER_W)], idx_v)
        pltpu.sync_copy(pos_hbm.at[pl.ds(0, S)], pos_v)
        out_base = wid * PER_W

        def seq_body(c, carry):
            g1 = pltpu.make_async_copy(
                tab_hbm.at[idx_v.at[2 * c]], buf_v.at[pl.ds(0, HALF)], sem)
            g2 = pltpu.make_async_copy(
                tab_hbm.at[idx_v.at[2 * c + 1]], buf_v.at[pl.ds(HALF, HALF)], sem)
            g1.start()
            g2.start()
            g1.wait()
            g2.wait()

            def add_row(r, carry2):
                for j in range(D // L):
                    sl = pl.ds(j * L, L)
                    plsc.addupdate(buf_v.at[r, sl], pos_v[r, sl])
                return carry2

            lax.fori_loop(0, S, add_row, 0)
            pltpu.sync_copy(buf_v, out_hbm.at[pl.ds(out_base + c * S, S)])
            return carry

        lax.fori_loop(0, SEQ_PER_W, seq_body, 0)

    return body(x2d, table, pos)


def kernel(x, embed_table, pos_table):
    B, S = x.shape
    D = embed_table.shape[1]
    x2d = x.reshape(B * S // (S // 2), S // 2)
    out = _sc_embed(x2d, embed_table, pos_table, B, S, D)
    return out.reshape(B, S, D)
